# baseline (speedup 1.0000x reference)
import numpy as np

import jax
import jax.numpy as jnp
from jax.experimental import pallas as pl
from jax.experimental.pallas import tpu as pltpu


def _blk_a(p_ref, wc_ref, bc_ref, w1_ref, b1_ref, w2_ref, b2_ref, o_ref):
    c = []
    for p in range(4):
        cols = [jnp.dot(p_ref[i, p], wc_ref[...],
                        preferred_element_type=jnp.float32)
                for i in range(4)]
        cp = jnp.concatenate(cols, axis=-1) + bc_ref[...]
        c.append(jnp.maximum(cp, 0.0))

    acc = jnp.zeros((64, 128), jnp.float32)
    for p in range(4):
        acc = acc + jnp.dot(w1_ref[p], c[p],
                            preferred_element_type=jnp.float32)
    h = jnp.maximum(acc + b1_ref[...], 0.0)

    out = None
    for p in range(4):
        att = jnp.dot(w2_ref[p], h, preferred_element_type=jnp.float32)
        att = att + b2_ref[:, pl.ds(p, 1)]
        y = c[p] * att
        out = y if out is None else jnp.maximum(out, y)
    o_ref[...] = out


def _blk_b(x_ref, wt_ref, bc_ref, mk_ref, w1_ref, b1_ref, w2_ref, b2_ref,
           e_ref, o_ref):
    acc = jnp.zeros((196, 256), jnp.float32)
    for t in range(25):
        di, dj = t // 5, t % 5
        s = (di - 2) * 14 + (dj - 2)
        xs = x_ref[pl.ds(30 + s, 196), :] * mk_ref[:, pl.ds(t, 1)]
        acc = acc + jnp.dot(xs, wt_ref[t],
                            preferred_element_type=jnp.float32)
    c = jnp.maximum(acc + bc_ref[...], 0.0)

    h = jnp.dot(w1_ref[...], c, preferred_element_type=jnp.float32)
    h = jnp.maximum(h + b1_ref[...], 0.0)
    att = jnp.dot(w2_ref[...], h, preferred_element_type=jnp.float32)
    att = att + b2_ref[...]
    y = c * att

    out = None
    for p in range(4):
        sel = jnp.dot(e_ref[p], y, preferred_element_type=jnp.float32)
        out = sel if out is None else jnp.maximum(out, sel)
    o_ref[...] = out


def _blk_c(x_ref, w1_ref, b1_ref, w2_ref, b2_ref, o_ref):
    h = jnp.dot(x_ref[...], w1_ref[...], preferred_element_type=jnp.float32)
    h = jnp.maximum(h + b1_ref[...], 0.0)
    o_ref[...] = (jnp.dot(h, w2_ref[...], preferred_element_type=jnp.float32)
                  + b2_ref[...])


def _phase_patches(x):
    B = x.shape[0]
    xp = jnp.pad(x, ((0, 0), (0, 0), (2, 2), (2, 2)))[:, 0]
    phases = []
    for a in range(2):
        for b in range(2):
            taps = []
            for di in range(5):
                for dj in range(5):
                    sl = xp[:, a + di:a + di + 28:2, b + dj:b + dj + 28:2]
                    taps.append(sl.reshape(B, 196))
            phases.append(jnp.stack(taps, axis=-1))
    return jnp.stack(phases, axis=1)


def _ca1_split(ca1_w1, ca1_w2, ca1_b2):
    w1r = ca1_w1.reshape(64, 28, 28)
    w2r = ca1_w2.reshape(28, 28, 64)
    b2r = ca1_b2.reshape(28, 28)
    w1a = jnp.stack([w1r[:, a::2, b::2].reshape(64, 196)
                     for a in range(2) for b in range(2)], axis=0)
    w2a = jnp.stack([w2r[a::2, b::2, :].reshape(196, 64)
                     for a in range(2) for b in range(2)], axis=0)
    b2a = jnp.stack([b2r[a::2, b::2].reshape(196)
                     for a in range(2) for b in range(2)], axis=1)
    return w1a, w2a, b2a


def _tap_masks():
    m = np.zeros((196, 25), np.float32)
    for t in range(25):
        dio, djo = t // 5 - 2, t % 5 - 2
        for idx in range(196):
            i, j = idx // 14, idx % 14
            if 0 <= i + dio < 14 and 0 <= j + djo < 14:
                m[idx, t] = 1.0
    return jnp.asarray(m)


def _pool_select():
    e = np.zeros((4, 49, 196), np.float32)
    for p in range(4):
        a, b = p // 2, p % 2
        for ih in range(7):
            for jw in range(7):
                e[p, ih * 7 + jw, (2 * ih + a) * 14 + (2 * jw + b)] = 1.0
    return jnp.asarray(e)


def _pp():
    return pltpu.CompilerParams(dimension_semantics=("parallel",))


def kernel(inputs, conv1_w, conv1_b, conv2_w, conv2_b, fc1_w, fc1_b, fc2_w,
           fc2_b, ca1_w1, ca1_b1, ca1_w2, ca1_b2, ca2_w1, ca2_b1, ca2_w2,
           ca2_b2):
    x = inputs.reshape(-1, 1, 28, 28).astype(jnp.float32)
    B = x.shape[0]
    G = B // 4

    wc1 = conv1_w.reshape(32, 25).T
    bc1 = jnp.tile(conv1_b.reshape(1, 32), (1, 4))
    w1a, w2a, b2a = _ca1_split(ca1_w1, ca1_w2, ca1_b2)
    b1a = ca1_b1.reshape(64, 1)

    wtb = conv2_w.transpose(2, 3, 1, 0).reshape(25, 32, 64)
    wt4 = jnp.einsum('tij,ab->taibj', wtb,
                     jnp.eye(4, dtype=jnp.float32)).reshape(25, 128, 256)
    bc2 = jnp.tile(conv2_b.reshape(1, 64), (1, 4))
    w1b = ca2_w1
    b1b = ca2_b1.reshape(64, 1)
    w2b = ca2_w2
    b2b = ca2_b2.reshape(196, 1)
    mkb = _tap_masks()
    esel = _pool_select()

    w1c = fc1_w.reshape(512, 64, 49).transpose(2, 1, 0).reshape(3136, 512)
    b1c = fc1_b.reshape(1, 512)
    w2c = jnp.pad(fc2_w.T, ((0, 0), (0, 118)))
    b2c = jnp.pad(fc2_b, (0, 118)).reshape(1, 128)

    patches = _phase_patches(x).reshape(G, 4, 4, 196, 25)
    fa = 2 * G * 4 * (4 * 196 * 25 * 32 + 64 * 196 * 128 + 196 * 64 * 128)
    ba = 4 * (patches.size + wc1.size + 128 + w1a.size + 64 + w2a.size
              + b2a.size + G * 196 * 128)
    a_out = pl.pallas_call(
        _blk_a,
        out_shape=jax.ShapeDtypeStruct((G, 196, 128), jnp.float32),
        grid=(G,),
        in_specs=[
            pl.BlockSpec((None, 4, 4, 196, 25), lambda i: (i, 0, 0, 0, 0)),
            pl.BlockSpec((25, 32), lambda i: (0, 0)),
            pl.BlockSpec((1, 128), lambda i: (0, 0)),
            pl.BlockSpec((4, 64, 196), lambda i: (0, 0, 0)),
            pl.BlockSpec((64, 1), lambda i: (0, 0)),
            pl.BlockSpec((4, 196, 64), lambda i: (0, 0, 0)),
            pl.BlockSpec((196, 4), lambda i: (0, 0)),
        ],
        out_specs=pl.BlockSpec((None, 196, 128), lambda i: (i, 0, 0)),
        compiler_params=_pp(),
        cost_estimate=pl.CostEstimate(flops=fa, transcendentals=0,
                                      bytes_accessed=ba),
    )(patches, wc1, bc1, w1a, b1a, w2a, b2a)

    xpad = jnp.pad(a_out, ((0, 0), (30, 30), (0, 0)))
    fb = 2 * G * (25 * 196 * 128 * 256 + 64 * 196 * 256 + 196 * 64 * 256
                  + 4 * 49 * 196 * 256)
    bb = 4 * (xpad.size + wt4.size + 256 + mkb.size + w1b.size + 64
              + w2b.size + 196 + esel.size + G * 49 * 256)
    b_out = pl.pallas_call(
        _blk_b,
        out_shape=jax.ShapeDtypeStruct((G, 49, 256), jnp.float32),
        grid=(G,),
        in_specs=[
            pl.BlockSpec((None, 256, 128), lambda i: (i, 0, 0)),
            pl.BlockSpec((25, 128, 256), lambda i: (0, 0, 0)),
            pl.BlockSpec((1, 256), lambda i: (0, 0)),
            pl.BlockSpec((196, 25), lambda i: (0, 0)),
            pl.BlockSpec((64, 196), lambda i: (0, 0)),
            pl.BlockSpec((64, 1), lambda i: (0, 0)),
            pl.BlockSpec((196, 64), lambda i: (0, 0)),
            pl.BlockSpec((196, 1), lambda i: (0, 0)),
            pl.BlockSpec((4, 49, 196), lambda i: (0, 0, 0)),
        ],
        out_specs=pl.BlockSpec((None, 49, 256), lambda i: (i, 0, 0)),
        compiler_params=_pp(),
        cost_estimate=pl.CostEstimate(flops=fb, transcendentals=0,
                                      bytes_accessed=bb),
    )(xpad, wt4, bc2, mkb, w1b, b1b, w2b, b2b, esel)

    flat = b_out.reshape(G, 49, 4, 64).transpose(0, 2, 1, 3).reshape(B, 3136)
    tb = min(B, 128)
    fc = 2 * B * (3136 * 512 + 512 * 128)
    bc = 4 * (flat.size + w1c.size + 512 + w2c.size + 128 + B * 128)
    logits = pl.pallas_call(
        _blk_c,
        out_shape=jax.ShapeDtypeStruct((B, 128), jnp.float32),
        grid=(pl.cdiv(B, tb),),
        in_specs=[
            pl.BlockSpec((tb, 3136), lambda i: (i, 0)),
            pl.BlockSpec((3136, 512), lambda i: (0, 0)),
            pl.BlockSpec((1, 512), lambda i: (0, 0)),
            pl.BlockSpec((512, 128), lambda i: (0, 0)),
            pl.BlockSpec((1, 128), lambda i: (0, 0)),
        ],
        out_specs=pl.BlockSpec((tb, 128), lambda i: (i, 0)),
        compiler_params=_pp(),
        cost_estimate=pl.CostEstimate(flops=fc, transcendentals=0,
                                      bytes_accessed=bc),
    )(flat, w1c, b1c, w2c, b2c)
    return logits[:, :10]


# speedup vs baseline: 6.4186x; 6.4186x over previous
import numpy as np

import jax
import jax.numpy as jnp
from jax.experimental import pallas as pl
from jax.experimental.pallas import tpu as pltpu


def _blk_a(p_ref, wc_ref, bc_ref, w1_ref, b1_ref, w2_ref, b2_ref, o_ref):
    cc = jnp.dot(p_ref[...], wc_ref[...], preferred_element_type=jnp.float32)
    cc = jnp.maximum(cc + bc_ref[...], 0.0)
    c = [cc[:, 128 * p:128 * (p + 1)] for p in range(4)]

    acc = jnp.zeros((64, 128), jnp.float32)
    for p in range(4):
        acc = acc + jnp.dot(w1_ref[p], c[p],
                            preferred_element_type=jnp.float32)
    h = jnp.maximum(acc + b1_ref[...], 0.0)

    out = None
    for p in range(4):
        att = jnp.dot(w2_ref[p], h, preferred_element_type=jnp.float32)
        att = att + b2_ref[:, pl.ds(p, 1)]
        y = c[p] * att
        out = y if out is None else jnp.maximum(out, y)
    o_ref[...] = out


def _blk_b(x_ref, wt_ref, bc_ref, mk_ref, w1_ref, b1_ref, w2_ref, b2_ref,
           e_ref, o_ref):
    acc = jnp.zeros((196, 256), jnp.float32)
    for t in range(25):
        di, dj = t // 5, t % 5
        s = (di - 2) * 14 + (dj - 2)
        xs = x_ref[pl.ds(30 + s, 196), :] * mk_ref[:, pl.ds(t, 1)]
        acc = acc + jnp.dot(xs, wt_ref[t],
                            preferred_element_type=jnp.float32)
    c = jnp.maximum(acc + bc_ref[...], 0.0)

    h = jnp.dot(w1_ref[...], c, preferred_element_type=jnp.float32)
    h = jnp.maximum(h + b1_ref[...], 0.0)
    att = jnp.dot(w2_ref[...], h, preferred_element_type=jnp.float32)
    att = att + b2_ref[...]
    y = c * att

    out = None
    for p in range(4):
        sel = jnp.dot(e_ref[p], y, preferred_element_type=jnp.float32)
        out = sel if out is None else jnp.maximum(out, sel)
    o_ref[...] = out


def _blk_c(x_ref, w1_ref, b1_ref, w2_ref, b2_ref, o_ref):
    h = jnp.dot(x_ref[...], w1_ref[...], preferred_element_type=jnp.float32)
    h = jnp.maximum(h + b1_ref[...], 0.0)
    o_ref[...] = (jnp.dot(h, w2_ref[...], preferred_element_type=jnp.float32)
                  + b2_ref[...])


def _phase_patches(x):
    B = x.shape[0]
    xp = jnp.pad(x, ((0, 0), (0, 0), (2, 2), (2, 2)))[:, 0]
    phases = []
    for a in range(2):
        for b in range(2):
            taps = []
            for di in range(5):
                for dj in range(5):
                    sl = xp[:, a + di:a + di + 28:2, b + dj:b + dj + 28:2]
                    taps.append(sl.reshape(B, 196))
            phases.append(jnp.stack(taps, axis=-1))
    return jnp.stack(phases, axis=1)


def _ca1_split(ca1_w1, ca1_w2, ca1_b2):
    w1r = ca1_w1.reshape(64, 28, 28)
    w2r = ca1_w2.reshape(28, 28, 64)
    b2r = ca1_b2.reshape(28, 28)
    w1a = jnp.stack([w1r[:, a::2, b::2].reshape(64, 196)
                     for a in range(2) for b in range(2)], axis=0)
    w2a = jnp.stack([w2r[a::2, b::2, :].reshape(196, 64)
                     for a in range(2) for b in range(2)], axis=0)
    b2a = jnp.stack([b2r[a::2, b::2].reshape(196)
                     for a in range(2) for b in range(2)], axis=1)
    return w1a, w2a, b2a


def _tap_masks():
    m = np.zeros((196, 25), np.float32)
    for t in range(25):
        dio, djo = t // 5 - 2, t % 5 - 2
        for idx in range(196):
            i, j = idx // 14, idx % 14
            if 0 <= i + dio < 14 and 0 <= j + djo < 14:
                m[idx, t] = 1.0
    return jnp.asarray(m)


def _pool_select():
    e = np.zeros((4, 49, 196), np.float32)
    for p in range(4):
        a, b = p // 2, p % 2
        for ih in range(7):
            for jw in range(7):
                e[p, ih * 7 + jw, (2 * ih + a) * 14 + (2 * jw + b)] = 1.0
    return jnp.asarray(e)


def _pp():
    return pltpu.CompilerParams(dimension_semantics=("parallel",))


def kernel(inputs, conv1_w, conv1_b, conv2_w, conv2_b, fc1_w, fc1_b, fc2_w,
           fc2_b, ca1_w1, ca1_b1, ca1_w2, ca1_b2, ca2_w1, ca2_b1, ca2_w2,
           ca2_b2):
    x = inputs.reshape(-1, 1, 28, 28).astype(jnp.float32)
    B = x.shape[0]
    G = B // 4

    wc1 = conv1_w.reshape(32, 25).T
    eye4 = jnp.eye(4, dtype=jnp.float32)
    wc16 = jnp.kron(eye4, jnp.kron(eye4, wc1))
    bc1 = jnp.tile(conv1_b.reshape(1, 32), (1, 16))
    w1a, w2a, b2a = _ca1_split(ca1_w1, ca1_w2, ca1_b2)
    b1a = ca1_b1.reshape(64, 1)

    wtb = conv2_w.transpose(2, 3, 1, 0).reshape(25, 32, 64)
    wt4 = jnp.einsum('tij,ab->taibj', wtb,
                     jnp.eye(4, dtype=jnp.float32)).reshape(25, 128, 256)
    bc2 = jnp.tile(conv2_b.reshape(1, 64), (1, 4))
    w1b = ca2_w1
    b1b = ca2_b1.reshape(64, 1)
    w2b = ca2_w2
    b2b = ca2_b2.reshape(196, 1)
    mkb = _tap_masks()
    esel = _pool_select()

    w1c = fc1_w.reshape(512, 64, 49).transpose(2, 1, 0).reshape(3136, 512)
    b1c = fc1_b.reshape(1, 512)
    w2c = jnp.pad(fc2_w.T, ((0, 0), (0, 118)))
    b2c = jnp.pad(fc2_b, (0, 118)).reshape(1, 128)

    patches = (_phase_patches(x).reshape(G, 4, 4, 196, 25)
               .transpose(0, 3, 2, 1, 4).reshape(G, 196, 400))
    fa = 2 * G * (196 * 400 * 512 + 4 * 64 * 196 * 128 + 4 * 196 * 64 * 128)
    ba = 4 * (patches.size + wc16.size + 512 + w1a.size + 64 + w2a.size
              + b2a.size + G * 196 * 128)
    a_out = pl.pallas_call(
        _blk_a,
        out_shape=jax.ShapeDtypeStruct((G, 196, 128), jnp.float32),
        grid=(G,),
        in_specs=[
            pl.BlockSpec((None, 196, 400), lambda i: (i, 0, 0)),
            pl.BlockSpec((400, 512), lambda i: (0, 0)),
            pl.BlockSpec((1, 512), lambda i: (0, 0)),
            pl.BlockSpec((4, 64, 196), lambda i: (0, 0, 0)),
            pl.BlockSpec((64, 1), lambda i: (0, 0)),
            pl.BlockSpec((4, 196, 64), lambda i: (0, 0, 0)),
            pl.BlockSpec((196, 4), lambda i: (0, 0)),
        ],
        out_specs=pl.BlockSpec((None, 196, 128), lambda i: (i, 0, 0)),
        compiler_params=_pp(),
        cost_estimate=pl.CostEstimate(flops=fa, transcendentals=0,
                                      bytes_accessed=ba),
    )(patches, wc16, bc1, w1a, b1a, w2a, b2a)

    xpad = jnp.pad(a_out, ((0, 0), (30, 30), (0, 0)))
    fb = 2 * G * (25 * 196 * 128 * 256 + 64 * 196 * 256 + 196 * 64 * 256
                  + 4 * 49 * 196 * 256)
    bb = 4 * (xpad.size + wt4.size + 256 + mkb.size + w1b.size + 64
              + w2b.size + 196 + esel.size + G * 49 * 256)
    b_out = pl.pallas_call(
        _blk_b,
        out_shape=jax.ShapeDtypeStruct((G, 49, 256), jnp.float32),
        grid=(G,),
        in_specs=[
            pl.BlockSpec((None, 256, 128), lambda i: (i, 0, 0)),
            pl.BlockSpec((25, 128, 256), lambda i: (0, 0, 0)),
            pl.BlockSpec((1, 256), lambda i: (0, 0)),
            pl.BlockSpec((196, 25), lambda i: (0, 0)),
            pl.BlockSpec((64, 196), lambda i: (0, 0)),
            pl.BlockSpec((64, 1), lambda i: (0, 0)),
            pl.BlockSpec((196, 64), lambda i: (0, 0)),
            pl.BlockSpec((196, 1), lambda i: (0, 0)),
            pl.BlockSpec((4, 49, 196), lambda i: (0, 0, 0)),
        ],
        out_specs=pl.BlockSpec((None, 49, 256), lambda i: (i, 0, 0)),
        compiler_params=_pp(),
        cost_estimate=pl.CostEstimate(flops=fb, transcendentals=0,
                                      bytes_accessed=bb),
    )(xpad, wt4, bc2, mkb, w1b, b1b, w2b, b2b, esel)

    flat = b_out.reshape(G, 49, 4, 64).transpose(0, 2, 1, 3).reshape(B, 3136)
    tb = min(B, 128)
    fc = 2 * B * (3136 * 512 + 512 * 128)
    bc = 4 * (flat.size + w1c.size + 512 + w2c.size + 128 + B * 128)
    logits = pl.pallas_call(
        _blk_c,
        out_shape=jax.ShapeDtypeStruct((B, 128), jnp.float32),
        grid=(pl.cdiv(B, tb),),
        in_specs=[
            pl.BlockSpec((tb, 3136), lambda i: (i, 0)),
            pl.BlockSpec((3136, 512), lambda i: (0, 0)),
            pl.BlockSpec((1, 512), lambda i: (0, 0)),
            pl.BlockSpec((512, 128), lambda i: (0, 0)),
            pl.BlockSpec((1, 128), lambda i: (0, 0)),
        ],
        out_specs=pl.BlockSpec((tb, 128), lambda i: (i, 0)),
        compiler_params=_pp(),
        cost_estimate=pl.CostEstimate(flops=fc, transcendentals=0,
                                      bytes_accessed=bc),
    )(flat, w1c, b1c, w2c, b2c)
    return logits[:, :10]
